# Initial kernel scaffold; baseline (speedup 1.0000x reference)
import jax
import jax.numpy as jnp
from jax.experimental import pallas as pl


def kernel(hidden_states, wq, wk, wv, wo, ln_in, ln_post_attn, ln_pre_ff, ln_post_ff, gate_w, w1, w2, w3):
    raise NotImplementedError("write your pallas kernel here")


# speedup vs baseline: 1.1628x; 1.1628x over previous
import functools

import jax
import jax.numpy as jnp
from jax import lax
from jax.experimental import pallas as pl
from jax.experimental.pallas import tpu as pltpu
from jax.experimental.pallas import tpu_sc as plsc

F32 = jnp.float32
BF16 = jnp.bfloat16
I32 = jnp.int32

EPS = 1e-6
SOFTCAP = 50.0
JITTER = 0.01
THETA = 10000.0
NEG = -1e30

BT = 256
BS = 512
NW = 32
SC_W = 16


def _rmsn(x, w):
    x32 = x.astype(jnp.float32)
    var = jnp.mean(x32 * x32, axis=-1, keepdims=True)
    x32 = x32 * jax.lax.rsqrt(var + EPS)
    return (x32 * (1.0 + w.astype(jnp.float32))).astype(x.dtype)


def _rot_half(x):
    x1 = x[..., : x.shape[-1] // 2]
    x2 = x[..., x.shape[-1] // 2:]
    return jnp.concatenate([-x2, x1], axis=-1)


def _mixer(scores, jitter_eps=JITTER):
    max_val = jnp.max(scores, axis=-1, keepdims=True)
    max_ind = jnp.argmax(scores, axis=-1, keepdims=True)
    factor = jnp.maximum(jnp.abs(scores), max_val)
    mask = ((max_val - scores) / factor) > 2.0 * jitter_eps
    masked_gates = jax.nn.softmax(jnp.where(mask, -jnp.inf, scores), axis=-1)
    mult1 = jnp.take_along_axis(masked_gates, max_ind, axis=-1)
    oh = jax.nn.one_hot(max_ind[..., 0], scores.shape[-1], dtype=jnp.bool_)
    masked_scores = jnp.where(oh, -jnp.inf, scores)
    max2 = jnp.max(masked_scores, axis=-1, keepdims=True)
    ind2 = jnp.argmax(masked_scores, axis=-1, keepdims=True)
    factor2 = jnp.maximum(jnp.abs(scores), max2)
    mask2 = ((max2 - scores) / factor2) > 2.0 * jitter_eps
    masked_gates2 = jax.nn.softmax(jnp.where(mask2, -jnp.inf, masked_scores),
                                   axis=-1)
    mult2 = jnp.take_along_axis(masked_gates2, ind2, axis=-1)
    multiplier = jnp.concatenate([mult1, mult2], axis=-1)
    selected = jnp.concatenate([max_ind, ind2], axis=-1)
    return multiplier, selected


def _prelude(hidden_states, wq, wk, wv, wo, ln_in, ln_post_attn, ln_pre_ff,
             gate_w):
    bn, sn, dm = hidden_states.shape
    dh = 128
    h = wq.shape[1] // dh
    kvh = wk.shape[1] // dh
    scaling = float(dh) ** -0.5
    residual = hidden_states
    x = _rmsn(hidden_states, ln_in)
    q = (x @ wq).reshape(bn, sn, h, dh).transpose(0, 2, 1, 3)
    k = (x @ wk).reshape(bn, sn, kvh, dh).transpose(0, 2, 1, 3)
    v = (x @ wv).reshape(bn, sn, kvh, dh).transpose(0, 2, 1, 3)
    pos = jnp.arange(sn)
    inv = 1.0 / (THETA ** (jnp.arange(0, dh, 2).astype(jnp.float32) / dh))
    freqs = pos[:, None].astype(jnp.float32) * inv[None, :]
    emb = jnp.concatenate([freqs, freqs], axis=-1)
    cos = jnp.cos(emb)[None, None]
    sin = jnp.sin(emb)[None, None]
    q = q * cos + _rot_half(q) * sin
    k = k * cos + _rot_half(k) * sin
    k = jnp.repeat(k, h // kvh, axis=1)
    v = jnp.repeat(v, h // kvh, axis=1)
    scores = jnp.einsum('bhqd,bhkd->bhqk', q, k) * scaling
    scores = jnp.tanh(scores / SOFTCAP) * SOFTCAP
    causal = pos[:, None] >= pos[None, :]
    scores = jnp.where(causal[None, None], scores, -1e30)
    probs = jax.nn.softmax(scores.astype(jnp.float32), axis=-1).astype(x.dtype)
    attn = jnp.einsum('bhqk,bhkd->bhqd', probs, v)
    attn = attn.transpose(0, 2, 1, 3).reshape(bn, sn, h * dh)
    attn = attn @ wo
    attn = _rmsn(attn, ln_post_attn)
    hidden = residual + attn
    x2 = _rmsn(hidden, ln_pre_ff)
    xt = x2.reshape(bn * sn, dm)
    router_logits = xt @ gate_w
    multiplier, selected = _mixer(router_logits)
    return hidden.reshape(bn * sn, dm), xt, multiplier, selected


def _dispatch_body(s0_ref, s1_ref, d0_ref, d1_ref, te_ref, lv_ref,
                   *, bt, nt, e):
    t = s0_ref.shape[0]
    lane = lax.broadcasted_iota(I32, (t, e), 1)
    is0 = lane == s0_ref[...]
    is1 = lane == s1_ref[...]

    oh0 = is0.astype(BF16)
    oh1 = is1.astype(BF16)
    ch = 256
    r0s, r1s = [], []
    for c in range(t // ch):
        rowio = lax.broadcasted_iota(I32, (ch, t), 0) + c * ch
        colio = lax.broadcasted_iota(I32, (ch, t), 1)
        tril = (colio < rowio).astype(BF16)
        r0s.append(lax.dot_general(
            tril, oh0, (((1,), (0,)), ((), ())), preferred_element_type=F32))
        r1s.append(lax.dot_general(
            tril, oh1, (((1,), (0,)), ((), ())), preferred_element_type=F32))
    r0 = jnp.concatenate(r0s, axis=0)
    r1 = jnp.concatenate(r1s, axis=0)

    tot0 = jnp.sum(is0.astype(F32), axis=0, keepdims=True)
    tot1 = jnp.sum(is1.astype(F32), axis=0, keepdims=True)
    gtot = tot0 + tot1
    padded = jnp.floor((gtot + (bt - 1)) * (1.0 / bt)) * bt
    eio = lax.broadcasted_iota(I32, (e, e), 0)
    ejo = lax.broadcasted_iota(I32, (e, e), 1)
    strict = (eio < ejo).astype(BF16)
    start = lax.dot_general(
        padded.astype(BF16), strict, (((1,), (0,)), ((), ())),
        preferred_element_type=F32,
    )

    d0 = jnp.sum(jnp.where(is0, start + r0, 0.0), axis=1, keepdims=True)
    d1 = jnp.sum(jnp.where(is1, start + tot0 + r1, 0.0), axis=1,
                 keepdims=True)
    d0_ref[...] = d0.astype(I32)
    d1_ref[...] = d1.astype(I32)

    ti = (lax.broadcasted_iota(I32, (1, nt), 1) * bt).astype(F32)
    acc = jnp.zeros((1, nt), F32)
    for j in range(e):
        acc += (ti >= start[0:1, j:j + 1]).astype(F32)
    te_ref[...] = (acc - 1.0).astype(I32)
    used = start[0:1, e - 1:e] + padded[0:1, e - 1:e]
    lv_ref[...] = (ti < used).astype(I32)


def _ffn1_body(te_ref, lv_ref, xs_ref, w1_ref, w3_ref, h_ref):
    i = pl.program_id(0)

    @pl.when(lv_ref[i] == 1)
    def _():
        x = xs_ref[...].astype(BF16)
        h1 = lax.dot_general(
            x, w1_ref[0], (((1,), (0,)), ((), ())), preferred_element_type=F32)
        h3 = lax.dot_general(
            x, w3_ref[0], (((1,), (0,)), ((), ())), preferred_element_type=F32)
        h_ref[...] = (jax.nn.gelu(h1, approximate=True) * h3).astype(BF16)


def _ffn2_body(te_ref, lv_ref, h_ref, w2_ref, eo_ref):
    i = pl.program_id(0)

    @pl.when(lv_ref[i] == 1)
    def _():
        eo_ref[...] = lax.dot_general(
            h_ref[...], w2_ref[0], (((1,), (0,)), ((), ())),
            preferred_element_type=F32)


def _final_body(g0_ref, g1_ref, m0_ref, m1_ref, res_ref, ln_ref, o_ref):
    moe = m0_ref[...] * g0_ref[...] + m1_ref[...] * g1_ref[...]
    var = jnp.mean(moe * moe, axis=-1, keepdims=True) + EPS
    r = lax.rsqrt(var)
    r = r * (1.5 - 0.5 * var * r * r)
    r = r * (1.5 - 0.5 * var * r * r)
    o_ref[...] = res_ref[...] + moe * r * (1.0 + ln_ref[...])


def _sc_dispatch(xf, idx3, nbuf):
    nw, c, w = idx3.shape
    t, d = xf.shape
    mesh = plsc.VectorSubcoreMesh(core_axis_name="c", subcore_axis_name="s")

    @functools.partial(
        pl.kernel,
        mesh=mesh,
        out_type=jax.ShapeDtypeStruct((nbuf, d), F32),
        scratch_types=[
            pltpu.VMEM((c, w), I32),
            pltpu.VMEM((w, d), F32),
        ],
    )
    def k(xf_hbm, idx_hbm, xs_hbm, idx_v, rows_v):
        wid = lax.axis_index("s") * 2 + lax.axis_index("c")
        grp = wid // 16
        tbase = (wid - grp * 16) * (c * w)
        pltpu.sync_copy(idx_hbm.at[wid], idx_v)
        for j in range(c):
            pltpu.sync_copy(xf_hbm.at[pl.ds(tbase + j * w, w)], rows_v)
            pltpu.sync_copy(rows_v, xs_hbm.at[idx_v.at[j]])

    return k(xf, idx3)


def _sc_combine(eo, idx3):
    nw, c, w = idx3.shape
    d = eo.shape[1]
    na = nw * c * w
    mesh = plsc.VectorSubcoreMesh(core_axis_name="c", subcore_axis_name="s")

    @functools.partial(
        pl.kernel,
        mesh=mesh,
        out_type=jax.ShapeDtypeStruct((na, d), F32),
        scratch_types=[
            pltpu.VMEM((c, w), I32),
            pltpu.VMEM((w, d), F32),
            pltpu.SemaphoreType.DMA,
        ],
    )
    def k(eo_hbm, idx_hbm, g_hbm, idx_v, rows_v, sem):
        wid = lax.axis_index("s") * 2 + lax.axis_index("c")
        base = wid * (c * w)
        pltpu.sync_copy(idx_hbm.at[wid], idx_v)
        for j in range(c):
            pltpu.async_copy(eo_hbm.at[idx_v.at[j]], rows_v, sem).wait()
            pltpu.sync_copy(rows_v, g_hbm.at[pl.ds(base + j * w, w)])

    return k(eo, idx3)


def kernel(hidden_states, wq, wk, wv, wo, ln_in, ln_post_attn, ln_pre_ff,
           ln_post_ff, gate_w, w1, w2, w3):
    bn, sn, dm = hidden_states.shape
    t = bn * sn
    e = gate_w.shape[1]
    ff = w1.shape[2]

    nt = (2 * t + (e - 1) * (BT - 1) + BT - 1) // BT + 1
    nbuf = nt * BT

    h2, xf, multiplier, selected = _prelude(
        hidden_states, wq, wk, wv, wo, ln_in, ln_post_attn, ln_pre_ff,
        gate_w)
    m0 = multiplier[:, 0:1]
    m1 = multiplier[:, 1:2]
    s0 = selected[:, 0:1].astype(I32)
    s1 = selected[:, 1:2].astype(I32)

    d0, d1, te, lv = pl.pallas_call(
        functools.partial(_dispatch_body, bt=BT, nt=nt, e=e),
        grid=(1,),
        in_specs=[
            pl.BlockSpec((t, 1), lambda i: (0, 0)),
            pl.BlockSpec((t, 1), lambda i: (0, 0)),
        ],
        out_specs=[
            pl.BlockSpec((t, 1), lambda i: (0, 0)),
            pl.BlockSpec((t, 1), lambda i: (0, 0)),
            pl.BlockSpec((1, nt), lambda i: (0, 0)),
            pl.BlockSpec((1, nt), lambda i: (0, 0)),
        ],
        out_shape=[
            jax.ShapeDtypeStruct((t, 1), I32),
            jax.ShapeDtypeStruct((t, 1), I32),
            jax.ShapeDtypeStruct((1, nt), I32),
            jax.ShapeDtypeStruct((1, nt), I32),
        ],
    )(s0, s1)

    idx3 = jnp.concatenate(
        [d0.reshape(t), d1.reshape(t)]).reshape(NW, (2 * t) // (NW * SC_W),
                                                SC_W)

    xs = _sc_dispatch(xf, idx3, nbuf)

    te_flat = te.reshape(nt)
    lv_flat = lv.reshape(nt)

    hbuf = pl.pallas_call(
        _ffn1_body,
        grid_spec=pltpu.PrefetchScalarGridSpec(
            num_scalar_prefetch=2,
            grid=(nt,),
            in_specs=[
                pl.BlockSpec((BT, dm), lambda i, te_r, lv_r: (i, 0)),
                pl.BlockSpec((1, dm, ff),
                             lambda i, te_r, lv_r: (te_r[i], 0, 0)),
                pl.BlockSpec((1, dm, ff),
                             lambda i, te_r, lv_r: (te_r[i], 0, 0)),
            ],
            out_specs=pl.BlockSpec((BT, ff), lambda i, te_r, lv_r: (i, 0)),
        ),
        out_shape=jax.ShapeDtypeStruct((nbuf, ff), BF16),
    )(te_flat, lv_flat, xs, w1.astype(BF16), w3.astype(BF16))

    eo = pl.pallas_call(
        _ffn2_body,
        grid_spec=pltpu.PrefetchScalarGridSpec(
            num_scalar_prefetch=2,
            grid=(nt,),
            in_specs=[
                pl.BlockSpec((BT, ff), lambda i, te_r, lv_r: (i, 0)),
                pl.BlockSpec((1, ff, dm),
                             lambda i, te_r, lv_r: (te_r[i], 0, 0)),
            ],
            out_specs=pl.BlockSpec((BT, dm), lambda i, te_r, lv_r: (i, 0)),
        ),
        out_shape=jax.ShapeDtypeStruct((nbuf, dm), F32),
    )(te_flat, lv_flat, hbuf, w2.astype(BF16))

    g = _sc_combine(eo, idx3)

    out = pl.pallas_call(
        _final_body,
        grid=(t // BS,),
        in_specs=[
            pl.BlockSpec((BS, dm), lambda i: (i, 0)),
            pl.BlockSpec((BS, dm), lambda i, _n=t // BS: (i + _n, 0)),
            pl.BlockSpec((BS, 1), lambda i: (i, 0)),
            pl.BlockSpec((BS, 1), lambda i: (i, 0)),
            pl.BlockSpec((BS, dm), lambda i: (i, 0)),
            pl.BlockSpec((1, dm), lambda i: (0, 0)),
        ],
        out_specs=pl.BlockSpec((BS, dm), lambda i: (i, 0)),
        out_shape=jax.ShapeDtypeStruct((t, dm), F32),
    )(g, g, m0, m1, h2, ln_post_ff.reshape(1, dm))

    return out.reshape(bn, sn, dm)


# speedup vs baseline: 1.1652x; 1.0020x over previous
import functools

import jax
import jax.numpy as jnp
from jax import lax
from jax.experimental import pallas as pl
from jax.experimental.pallas import tpu as pltpu
from jax.experimental.pallas import tpu_sc as plsc

F32 = jnp.float32
BF16 = jnp.bfloat16
I32 = jnp.int32

EPS = 1e-6
SOFTCAP = 50.0
JITTER = 0.01
THETA = 10000.0
NEG = -1e30

BT = 256
BS = 512
NW = 32
SC_W = 16


def _rmsn(x, w):
    x32 = x.astype(jnp.float32)
    var = jnp.mean(x32 * x32, axis=-1, keepdims=True)
    x32 = x32 * jax.lax.rsqrt(var + EPS)
    return (x32 * (1.0 + w.astype(jnp.float32))).astype(x.dtype)


def _rot_half(x):
    x1 = x[..., : x.shape[-1] // 2]
    x2 = x[..., x.shape[-1] // 2:]
    return jnp.concatenate([-x2, x1], axis=-1)


def _mixer(scores, jitter_eps=JITTER):
    max_val = jnp.max(scores, axis=-1, keepdims=True)
    max_ind = jnp.argmax(scores, axis=-1, keepdims=True)
    factor = jnp.maximum(jnp.abs(scores), max_val)
    mask = ((max_val - scores) / factor) > 2.0 * jitter_eps
    masked_gates = jax.nn.softmax(jnp.where(mask, -jnp.inf, scores), axis=-1)
    mult1 = jnp.take_along_axis(masked_gates, max_ind, axis=-1)
    oh = jax.nn.one_hot(max_ind[..., 0], scores.shape[-1], dtype=jnp.bool_)
    masked_scores = jnp.where(oh, -jnp.inf, scores)
    max2 = jnp.max(masked_scores, axis=-1, keepdims=True)
    ind2 = jnp.argmax(masked_scores, axis=-1, keepdims=True)
    factor2 = jnp.maximum(jnp.abs(scores), max2)
    mask2 = ((max2 - scores) / factor2) > 2.0 * jitter_eps
    masked_gates2 = jax.nn.softmax(jnp.where(mask2, -jnp.inf, masked_scores),
                                   axis=-1)
    mult2 = jnp.take_along_axis(masked_gates2, ind2, axis=-1)
    multiplier = jnp.concatenate([mult1, mult2], axis=-1)
    selected = jnp.concatenate([max_ind, ind2], axis=-1)
    return multiplier, selected


def _prelude(hidden_states, wq, wk, wv, wo, ln_in, ln_post_attn, ln_pre_ff,
             gate_w):
    bn, sn, dm = hidden_states.shape
    dh = 128
    h = wq.shape[1] // dh
    kvh = wk.shape[1] // dh
    scaling = float(dh) ** -0.5
    residual = hidden_states
    x = _rmsn(hidden_states, ln_in)
    q = (x @ wq).reshape(bn, sn, h, dh).transpose(0, 2, 1, 3)
    k = (x @ wk).reshape(bn, sn, kvh, dh).transpose(0, 2, 1, 3)
    v = (x @ wv).reshape(bn, sn, kvh, dh).transpose(0, 2, 1, 3)
    pos = jnp.arange(sn)
    inv = 1.0 / (THETA ** (jnp.arange(0, dh, 2).astype(jnp.float32) / dh))
    freqs = pos[:, None].astype(jnp.float32) * inv[None, :]
    emb = jnp.concatenate([freqs, freqs], axis=-1)
    cos = jnp.cos(emb)[None, None]
    sin = jnp.sin(emb)[None, None]
    q = q * cos + _rot_half(q) * sin
    k = k * cos + _rot_half(k) * sin
    k = jnp.repeat(k, h // kvh, axis=1)
    v = jnp.repeat(v, h // kvh, axis=1)
    scores = jnp.einsum('bhqd,bhkd->bhqk', q, k) * scaling
    scores = jnp.tanh(scores / SOFTCAP) * SOFTCAP
    causal = pos[:, None] >= pos[None, :]
    scores = jnp.where(causal[None, None], scores, -1e30)
    probs = jax.nn.softmax(scores.astype(jnp.float32), axis=-1).astype(x.dtype)
    attn = jnp.einsum('bhqk,bhkd->bhqd', probs, v)
    attn = attn.transpose(0, 2, 1, 3).reshape(bn, sn, h * dh)
    attn = attn @ wo
    attn = _rmsn(attn, ln_post_attn)
    hidden = residual + attn
    x2 = _rmsn(hidden, ln_pre_ff)
    xt = x2.reshape(bn * sn, dm)
    router_logits = xt @ gate_w
    multiplier, selected = _mixer(router_logits)
    return hidden.reshape(bn * sn, dm), xt, multiplier, selected


def _dispatch_body(s0_ref, s1_ref, d0_ref, d1_ref, te_ref, lv_ref,
                   *, bt, nt, e):
    t = s0_ref.shape[0]
    lane = lax.broadcasted_iota(I32, (t, e), 1)
    is0 = lane == s0_ref[...]
    is1 = lane == s1_ref[...]

    oh0 = is0.astype(BF16)
    oh1 = is1.astype(BF16)
    ch = 256
    r0s, r1s = [], []
    for c in range(t // ch):
        rowio = lax.broadcasted_iota(I32, (ch, t), 0) + c * ch
        colio = lax.broadcasted_iota(I32, (ch, t), 1)
        tril = (colio < rowio).astype(BF16)
        r0s.append(lax.dot_general(
            tril, oh0, (((1,), (0,)), ((), ())), preferred_element_type=F32))
        r1s.append(lax.dot_general(
            tril, oh1, (((1,), (0,)), ((), ())), preferred_element_type=F32))
    r0 = jnp.concatenate(r0s, axis=0)
    r1 = jnp.concatenate(r1s, axis=0)

    tot0 = jnp.sum(is0.astype(F32), axis=0, keepdims=True)
    tot1 = jnp.sum(is1.astype(F32), axis=0, keepdims=True)
    gtot = tot0 + tot1
    padded = jnp.floor((gtot + (bt - 1)) * (1.0 / bt)) * bt
    eio = lax.broadcasted_iota(I32, (e, e), 0)
    ejo = lax.broadcasted_iota(I32, (e, e), 1)
    strict = (eio < ejo).astype(BF16)
    start = lax.dot_general(
        padded.astype(BF16), strict, (((1,), (0,)), ((), ())),
        preferred_element_type=F32,
    )

    d0 = jnp.sum(jnp.where(is0, start + r0, 0.0), axis=1, keepdims=True)
    d1 = jnp.sum(jnp.where(is1, start + tot0 + r1, 0.0), axis=1,
                 keepdims=True)
    d0_ref[...] = d0.astype(I32)
    d1_ref[...] = d1.astype(I32)

    ti = (lax.broadcasted_iota(I32, (1, nt), 1) * bt).astype(F32)
    acc = jnp.zeros((1, nt), F32)
    for j in range(e):
        acc += (ti >= start[0:1, j:j + 1]).astype(F32)
    te_ref[...] = (acc - 1.0).astype(I32)
    used = start[0:1, e - 1:e] + padded[0:1, e - 1:e]
    lv_ref[...] = (ti < used).astype(I32)


def _ffn1_body(te_ref, lv_ref, xs_ref, w1_ref, w3_ref, h_ref):
    i = pl.program_id(0)

    @pl.when(lv_ref[i] == 1)
    def _():
        x = xs_ref[...].astype(BF16)
        h1 = lax.dot_general(
            x, w1_ref[0], (((1,), (0,)), ((), ())), preferred_element_type=F32)
        h3 = lax.dot_general(
            x, w3_ref[0], (((1,), (0,)), ((), ())), preferred_element_type=F32)
        h_ref[...] = (jax.nn.gelu(h1, approximate=True) * h3).astype(BF16)


def _ffn2_body(te_ref, lv_ref, h_ref, w2_ref, eo_ref):
    i = pl.program_id(0)

    @pl.when(lv_ref[i] == 1)
    def _():
        eo_ref[...] = lax.dot_general(
            h_ref[...], w2_ref[0], (((1,), (0,)), ((), ())),
            preferred_element_type=F32)


def _final_body(g0_ref, g1_ref, m0_ref, m1_ref, res_ref, ln_ref, o_ref):
    moe = m0_ref[...] * g0_ref[...] + m1_ref[...] * g1_ref[...]
    var = jnp.mean(moe * moe, axis=-1, keepdims=True) + EPS
    r = lax.rsqrt(var)
    r = r * (1.5 - 0.5 * var * r * r)
    r = r * (1.5 - 0.5 * var * r * r)
    o_ref[...] = res_ref[...] + moe * r * (1.0 + ln_ref[...])


def _sc_dispatch(xf, idx3, nbuf):
    nw, c, w = idx3.shape
    t, d = xf.shape
    mesh = plsc.VectorSubcoreMesh(core_axis_name="c", subcore_axis_name="s")

    @functools.partial(
        pl.kernel,
        mesh=mesh,
        out_type=jax.ShapeDtypeStruct((nbuf, d), F32),
        scratch_types=[
            pltpu.VMEM((c, w), I32),
            pltpu.VMEM((w, d), F32),
            pltpu.VMEM((w, d), F32),
            pltpu.SemaphoreType.DMA,
            pltpu.SemaphoreType.DMA,
        ],
    )
    def k(xf_hbm, idx_hbm, xs_hbm, idx_v, rows_a, rows_b, sem_a, sem_b):
        wid = lax.axis_index("s") * 2 + lax.axis_index("c")
        grp = wid // 16
        tbase = (wid - grp * 16) * (c * w)
        pltpu.sync_copy(idx_hbm.at[wid], idx_v)
        bufs = [rows_a, rows_b]
        sems = [sem_a, sem_b]
        pltpu.async_copy(xf_hbm.at[pl.ds(tbase, w)], rows_a, sem_a)
        for j in range(c):
            cur, nxt = bufs[j % 2], bufs[(j + 1) % 2]
            pltpu.make_async_copy(
                xf_hbm.at[pl.ds(tbase + j * w, w)], cur, sems[j % 2]).wait()
            if j + 1 < c:
                pltpu.async_copy(
                    xf_hbm.at[pl.ds(tbase + (j + 1) * w, w)], nxt,
                    sems[(j + 1) % 2])
            pltpu.sync_copy(cur, xs_hbm.at[idx_v.at[j]])

    return k(xf, idx3)


def _sc_combine(eo, idx3):
    nw, c, w = idx3.shape
    d = eo.shape[1]
    na = nw * c * w
    mesh = plsc.VectorSubcoreMesh(core_axis_name="c", subcore_axis_name="s")

    @functools.partial(
        pl.kernel,
        mesh=mesh,
        out_type=jax.ShapeDtypeStruct((na, d), F32),
        scratch_types=[
            pltpu.VMEM((c, w), I32),
            pltpu.VMEM((w, d), F32),
            pltpu.VMEM((w, d), F32),
            pltpu.SemaphoreType.DMA,
            pltpu.SemaphoreType.DMA,
        ],
    )
    def k(eo_hbm, idx_hbm, g_hbm, idx_v, rows_a, rows_b, sem_a, sem_b):
        wid = lax.axis_index("s") * 2 + lax.axis_index("c")
        base = wid * (c * w)
        pltpu.sync_copy(idx_hbm.at[wid], idx_v)
        bufs = [rows_a, rows_b]
        sems = [sem_a, sem_b]
        pltpu.async_copy(eo_hbm.at[idx_v.at[0]], rows_a, sem_a)
        for j in range(c):
            cur, nxt = bufs[j % 2], bufs[(j + 1) % 2]
            pltpu.make_async_copy(
                eo_hbm.at[idx_v.at[j]], cur, sems[j % 2]).wait()
            if j + 1 < c:
                pltpu.async_copy(eo_hbm.at[idx_v.at[j + 1]], nxt,
                                 sems[(j + 1) % 2])
            pltpu.sync_copy(cur, g_hbm.at[pl.ds(base + j * w, w)])

    return k(eo, idx3)


def kernel(hidden_states, wq, wk, wv, wo, ln_in, ln_post_attn, ln_pre_ff,
           ln_post_ff, gate_w, w1, w2, w3):
    bn, sn, dm = hidden_states.shape
    t = bn * sn
    e = gate_w.shape[1]
    ff = w1.shape[2]

    nt = (2 * t + (e - 1) * (BT - 1) + BT - 1) // BT + 1
    nbuf = nt * BT

    h2, xf, multiplier, selected = _prelude(
        hidden_states, wq, wk, wv, wo, ln_in, ln_post_attn, ln_pre_ff,
        gate_w)
    m0 = multiplier[:, 0:1]
    m1 = multiplier[:, 1:2]
    s0 = selected[:, 0:1].astype(I32)
    s1 = selected[:, 1:2].astype(I32)

    d0, d1, te, lv = pl.pallas_call(
        functools.partial(_dispatch_body, bt=BT, nt=nt, e=e),
        grid=(1,),
        in_specs=[
            pl.BlockSpec((t, 1), lambda i: (0, 0)),
            pl.BlockSpec((t, 1), lambda i: (0, 0)),
        ],
        out_specs=[
            pl.BlockSpec((t, 1), lambda i: (0, 0)),
            pl.BlockSpec((t, 1), lambda i: (0, 0)),
            pl.BlockSpec((1, nt), lambda i: (0, 0)),
            pl.BlockSpec((1, nt), lambda i: (0, 0)),
        ],
        out_shape=[
            jax.ShapeDtypeStruct((t, 1), I32),
            jax.ShapeDtypeStruct((t, 1), I32),
            jax.ShapeDtypeStruct((1, nt), I32),
            jax.ShapeDtypeStruct((1, nt), I32),
        ],
    )(s0, s1)

    idx3 = jnp.concatenate(
        [d0.reshape(t), d1.reshape(t)]).reshape(NW, (2 * t) // (NW * SC_W),
                                                SC_W)

    xs = _sc_dispatch(xf, idx3, nbuf)

    te_flat = te.reshape(nt)
    lv_flat = lv.reshape(nt)

    hbuf = pl.pallas_call(
        _ffn1_body,
        grid_spec=pltpu.PrefetchScalarGridSpec(
            num_scalar_prefetch=2,
            grid=(nt,),
            in_specs=[
                pl.BlockSpec((BT, dm), lambda i, te_r, lv_r: (i, 0)),
                pl.BlockSpec((1, dm, ff),
                             lambda i, te_r, lv_r: (te_r[i], 0, 0)),
                pl.BlockSpec((1, dm, ff),
                             lambda i, te_r, lv_r: (te_r[i], 0, 0)),
            ],
            out_specs=pl.BlockSpec((BT, ff), lambda i, te_r, lv_r: (i, 0)),
        ),
        out_shape=jax.ShapeDtypeStruct((nbuf, ff), BF16),
    )(te_flat, lv_flat, xs, w1.astype(BF16), w3.astype(BF16))

    eo = pl.pallas_call(
        _ffn2_body,
        grid_spec=pltpu.PrefetchScalarGridSpec(
            num_scalar_prefetch=2,
            grid=(nt,),
            in_specs=[
                pl.BlockSpec((BT, ff), lambda i, te_r, lv_r: (i, 0)),
                pl.BlockSpec((1, ff, dm),
                             lambda i, te_r, lv_r: (te_r[i], 0, 0)),
            ],
            out_specs=pl.BlockSpec((BT, dm), lambda i, te_r, lv_r: (i, 0)),
        ),
        out_shape=jax.ShapeDtypeStruct((nbuf, dm), F32),
    )(te_flat, lv_flat, hbuf, w2.astype(BF16))

    g = _sc_combine(eo, idx3)

    out = pl.pallas_call(
        _final_body,
        grid=(t // BS,),
        in_specs=[
            pl.BlockSpec((BS, dm), lambda i: (i, 0)),
            pl.BlockSpec((BS, dm), lambda i, _n=t // BS: (i + _n, 0)),
            pl.BlockSpec((BS, 1), lambda i: (i, 0)),
            pl.BlockSpec((BS, 1), lambda i: (i, 0)),
            pl.BlockSpec((BS, dm), lambda i: (i, 0)),
            pl.BlockSpec((1, dm), lambda i: (0, 0)),
        ],
        out_specs=pl.BlockSpec((BS, dm), lambda i: (i, 0)),
        out_shape=jax.ShapeDtypeStruct((t, dm), F32),
    )(g, g, m0, m1, h2, ln_post_ff.reshape(1, dm))

    return out.reshape(bn, sn, dm)
